# Initial kernel scaffold; baseline (speedup 1.0000x reference)
import jax
import jax.numpy as jnp
from jax.experimental import pallas as pl


def kernel(x, edge_index, edge_attr, eW1, eb1, eW2, eb2, eW3, eb3, eg, ebt, nW1, nb1, nW2, nb2, nW3, nb3, ng, nbt):
    raise NotImplementedError("write your pallas kernel here")


# speedup vs baseline: 2.1721x; 2.1721x over previous
import functools

import jax
import jax.numpy as jnp
from jax import lax
from jax.experimental import pallas as pl
from jax.experimental.pallas import tpu as pltpu
from jax.experimental.pallas import tpu_sc as plsc

N_ = 10000
E_ = 160000
D_ = 128
H_ = 128
NBLK = 9

NC = 2
NS = 16
NW = NC * NS

CH = 128
PER_W = 5120
E_PAD = NW * PER_W
NCH = PER_W // CH
N_PAD = 10240
ROWS_PER_TILE = N_PAD // NS

_SC_MESH = plsc.VectorSubcoreMesh(
    core_axis_name="c", subcore_axis_name="s", num_cores=NC, num_subcores=NS)


def _sc_gather_body(pa, pb, src_h, dst_h, outa, outb,
                    src_v, dst_v, ba0, ba1, bb0, bb1,
                    sga0, sga1, sgb0, sgb1, swa0, swa1, swb0, swb1):
    cid = lax.axis_index("c")
    sid = lax.axis_index("s")
    wid = sid * NC + cid
    base = wid * PER_W
    pltpu.sync_copy(src_h.at[pl.ds(base, PER_W)], src_v)
    pltpu.sync_copy(dst_h.at[pl.ds(base, PER_W)], dst_v)
    ba = (ba0, ba1)
    bb = (bb0, bb1)
    sga = (sga0, sga1)
    sgb = (sgb0, sgb1)
    swa = (swa0, swa1)
    swb = (swb0, swb1)

    def start_gather(j, b):
        pltpu.async_copy(pa.at[src_v.at[pl.ds(j * CH, CH)]], ba[b], sga[b])
        pltpu.async_copy(pb.at[dst_v.at[pl.ds(j * CH, CH)]], bb[b], sgb[b])

    def wait_gather(b):
        pltpu.make_async_copy(pa.at[src_v.at[pl.ds(0, CH)]], ba[b], sga[b]).wait()
        pltpu.make_async_copy(pb.at[dst_v.at[pl.ds(0, CH)]], bb[b], sgb[b]).wait()

    def start_write(j, b):
        pltpu.async_copy(ba[b], outa.at[pl.ds(base + j * CH, CH)], swa[b])
        pltpu.async_copy(bb[b], outb.at[pl.ds(base + j * CH, CH)], swb[b])

    def wait_write(b):
        pltpu.make_async_copy(ba[b], outa.at[pl.ds(base, CH)], swa[b]).wait()
        pltpu.make_async_copy(bb[b], outb.at[pl.ds(base, CH)], swb[b]).wait()

    start_gather(0, 0)

    def outer(j2, carry):
        for b in range(2):
            j = 2 * j2 + b

            @pl.when(j >= 1)
            def _():
                wait_write(1 - b)

            @pl.when(j + 1 < NCH)
            def _():
                start_gather(j + 1, 1 - b)

            wait_gather(b)
            start_write(j, b)
        return carry

    lax.fori_loop(0, NCH // 2, outer, 0)
    wait_write(1)


_gather_call = pl.kernel(
    _sc_gather_body,
    out_type=[jax.ShapeDtypeStruct((E_PAD, H_), jnp.float32),
              jax.ShapeDtypeStruct((E_PAD, H_), jnp.float32)],
    mesh=_SC_MESH,
    scratch_types=[
        pltpu.VMEM((PER_W,), jnp.int32),
        pltpu.VMEM((PER_W,), jnp.int32),
        pltpu.VMEM((CH, H_), jnp.float32),
        pltpu.VMEM((CH, H_), jnp.float32),
        pltpu.VMEM((CH, H_), jnp.float32),
        pltpu.VMEM((CH, H_), jnp.float32),
    ] + [pltpu.SemaphoreType.DMA] * 8,
)


def _sc_scatter_body(ea_h, dst3_h, zeros_h, out_h,
                     idx_v, rb0, rb1, acc, sl0, sl1):
    cid = lax.axis_index("c")
    sid = lax.axis_index("s")
    wid = sid * NC + cid
    base = wid * PER_W
    rows = pl.ds(sid * ROWS_PER_TILE, ROWS_PER_TILE)
    pltpu.sync_copy(zeros_h.at[rows], acc.at[rows])
    pltpu.sync_copy(dst3_h.at[wid], idx_v)
    plsc.subcore_barrier()
    rb = (rb0, rb1)
    sl = (sl0, sl1)

    def start_load(j, b):
        pltpu.async_copy(ea_h.at[pl.ds(base + j * CH, CH)], rb[b], sl[b])

    def wait_load(b):
        pltpu.make_async_copy(ea_h.at[pl.ds(base, CH)], rb[b], sl[b]).wait()

    start_load(0, 0)

    def outer(j2, carry):
        for b in range(2):
            j = 2 * j2 + b

            @pl.when(j + 1 < NCH)
            def _():
                start_load(j + 1, 1 - b)

            wait_load(b)
            pltpu.sync_copy(rb[b], acc.at[idx_v.at[j]], add=True)
        return carry

    lax.fori_loop(0, NCH // 2, outer, 0)
    plsc.subcore_barrier()
    pltpu.sync_copy(acc.at[rows], out_h.at[cid, rows])


_scatter_call = pl.kernel(
    _sc_scatter_body,
    out_type=jax.ShapeDtypeStruct((NC, N_PAD, D_), jnp.float32),
    mesh=_SC_MESH,
    scratch_types=[
        pltpu.VMEM((NCH, CH), jnp.int32),
        pltpu.VMEM((CH, D_), jnp.float32),
        pltpu.VMEM((CH, D_), jnp.float32),
        pltpu.VMEM_SHARED((N_PAD, D_), jnp.float32),
        pltpu.SemaphoreType.DMA,
        pltpu.SemaphoreType.DMA,
    ],
)


TN = 1000
TE = 1024


def _proj_body(x, w1a, w1b, outa, outb):
    xb = x[...]
    outa[...] = jnp.dot(xb, w1a[...], preferred_element_type=jnp.float32)
    outb[...] = jnp.dot(xb, w1b[...], preferred_element_type=jnp.float32)


@jax.jit
def _proj(x, w1a, w1b):
    return pl.pallas_call(
        _proj_body,
        grid=(N_ // TN,),
        in_specs=[
            pl.BlockSpec((TN, D_), lambda i: (i, 0)),
            pl.BlockSpec((D_, H_), lambda i: (0, 0)),
            pl.BlockSpec((D_, H_), lambda i: (0, 0)),
        ],
        out_specs=[pl.BlockSpec((TN, H_), lambda i: (i, 0)),
                   pl.BlockSpec((TN, H_), lambda i: (i, 0))],
        out_shape=[jax.ShapeDtypeStruct((N_, H_), jnp.float32),
                   jax.ShapeDtypeStruct((N_, H_), jnp.float32)],
    )(x, w1a, w1b)


def _ln_2d(h, g, bt):
    m = jnp.mean(h, axis=-1, keepdims=True)
    v = jnp.mean((h - m) ** 2, axis=-1, keepdims=True)
    return (h - m) * lax.rsqrt(v + 1e-5) * g + bt


def _edge_body(ga, gb, ea, w1c, w2, w3, prm, out):
    p = prm[...]
    b1 = p[0:1, :]
    b2 = p[1:2, :]
    b3 = p[2:3, :]
    g = p[3:4, :]
    bt = p[4:5, :]
    eab = ea[...]
    h = ga[...] + gb[...] + jnp.dot(eab, w1c[...],
                                    preferred_element_type=jnp.float32) + b1
    h = jnp.maximum(h, 0.0)
    h = jnp.maximum(jnp.dot(h, w2[...], preferred_element_type=jnp.float32) + b2, 0.0)
    h = jnp.dot(h, w3[...], preferred_element_type=jnp.float32) + b3
    ln = _ln_2d(h, g, bt)
    row = TE * pl.program_id(0) + lax.broadcasted_iota(jnp.int32, (TE, 1), 0)
    out[...] = jnp.where(row < E_, eab + ln, 0.0)


@jax.jit
def _edge(ga, gb, ea, w1c, w2, w3, prm):
    return pl.pallas_call(
        _edge_body,
        grid=(E_PAD // TE,),
        in_specs=[
            pl.BlockSpec((TE, H_), lambda i: (i, 0)),
            pl.BlockSpec((TE, H_), lambda i: (i, 0)),
            pl.BlockSpec((TE, D_), lambda i: (i, 0)),
            pl.BlockSpec((D_, H_), lambda i: (0, 0)),
            pl.BlockSpec((H_, H_), lambda i: (0, 0)),
            pl.BlockSpec((H_, D_), lambda i: (0, 0)),
            pl.BlockSpec((8, 128), lambda i: (0, 0)),
        ],
        out_specs=pl.BlockSpec((TE, D_), lambda i: (i, 0)),
        out_shape=jax.ShapeDtypeStruct((E_PAD, D_), jnp.float32),
    )(ga, gb, ea, w1c, w2, w3, prm)


def _node_body(x, a0, a1, w1a, w1b, w2, w3, prm, out):
    p = prm[...]
    b1 = p[0:1, :]
    b2 = p[1:2, :]
    b3 = p[2:3, :]
    g = p[3:4, :]
    bt = p[4:5, :]
    xb = x[...]
    agg = a0[...] + a1[...]
    h = (jnp.dot(xb, w1a[...], preferred_element_type=jnp.float32)
         + jnp.dot(agg, w1b[...], preferred_element_type=jnp.float32) + b1)
    h = jnp.maximum(h, 0.0)
    h = jnp.maximum(jnp.dot(h, w2[...], preferred_element_type=jnp.float32) + b2, 0.0)
    h = jnp.dot(h, w3[...], preferred_element_type=jnp.float32) + b3
    out[...] = xb + _ln_2d(h, g, bt)


@jax.jit
def _node(x, a0, a1, w1a, w1b, w2, w3, prm):
    return pl.pallas_call(
        _node_body,
        grid=(N_ // TN,),
        in_specs=[
            pl.BlockSpec((TN, D_), lambda i: (i, 0)),
            pl.BlockSpec((TN, D_), lambda i: (i, 0)),
            pl.BlockSpec((TN, D_), lambda i: (i, 0)),
            pl.BlockSpec((D_, H_), lambda i: (0, 0)),
            pl.BlockSpec((D_, H_), lambda i: (0, 0)),
            pl.BlockSpec((H_, H_), lambda i: (0, 0)),
            pl.BlockSpec((H_, D_), lambda i: (0, 0)),
            pl.BlockSpec((8, 128), lambda i: (0, 0)),
        ],
        out_specs=pl.BlockSpec((TN, D_), lambda i: (i, 0)),
        out_shape=jax.ShapeDtypeStruct((N_, D_), jnp.float32),
    )(x, a0, a1, w1a, w1b, w2, w3, prm)


def _pack_params(b1, b2, b3, g, bt):
    p = jnp.stack([b1, b2, b3, g, bt], axis=1)
    return jnp.pad(p, ((0, 0), (0, 3), (0, 0)))


def kernel(x, edge_index, edge_attr, eW1, eb1, eW2, eb2, eW3, eb3, eg, ebt,
           nW1, nb1, nW2, nb2, nW3, nb3, ng, nbt):
    pad = E_PAD - E_
    src_p = jnp.pad(edge_index[0], (0, pad))
    dst_p = jnp.pad(edge_index[1], (0, pad))
    dst3 = dst_p.reshape(NW, NCH, CH)
    ea = jnp.pad(edge_attr, ((0, pad), (0, 0)))
    zeros_nd = jnp.zeros((N_PAD, D_), jnp.float32)

    eW1a = eW1[:, :D_, :]
    eW1b = eW1[:, D_:2 * D_, :]
    eW1c = eW1[:, 2 * D_:, :]
    nW1a = nW1[:, :D_, :]
    nW1b = nW1[:, D_:, :]
    eprm = _pack_params(eb1, eb2, eb3, eg, ebt)
    nprm = _pack_params(nb1, nb2, nb3, ng, nbt)

    for i in range(NBLK):
        pa, pb = _proj(x, eW1a[i], eW1b[i])
        ga, gb = _gather_call(pa, pb, src_p, dst_p)
        ea = _edge(ga, gb, ea, eW1c[i], eW2[i], eW3[i], eprm[i])
        parts = _scatter_call(ea, dst3, zeros_nd)
        x = _node(x, parts[0, :N_], parts[1, :N_], nW1a[i], nW1b[i], nW2[i], nW3[i], nprm[i])
    return x


# speedup vs baseline: 3.7958x; 1.7475x over previous
import functools

import jax
import jax.numpy as jnp
from jax import lax
from jax.experimental import pallas as pl
from jax.experimental.pallas import tpu as pltpu
from jax.experimental.pallas import tpu_sc as plsc

N_ = 10000
E_ = 160000
D_ = 128
H_ = 128
HP = H_ // 2
NBLK = 9

NC = 2
NS = 16
NW = NC * NS

CH = 64
PER_W = 5120
E_PAD = NW * PER_W
NCH = PER_W // CH
N_PAD = 10240
ROWS_PER_TILE = N_PAD // NS

_SC_MESH = plsc.VectorSubcoreMesh(
    core_axis_name="c", subcore_axis_name="s", num_cores=NC, num_subcores=NS)


def _sc_gather_body(t_h, src_h, dst_h, out_h,
                    src_v, dst_v, ba0, ba1, bb0, bb1, tsp,
                    sga0, sga1, sgb0, sgb1, swo0, swo1):
    cid = lax.axis_index("c")
    sid = lax.axis_index("s")
    wid = sid * NC + cid
    base = wid * PER_W
    srows = pl.ds(sid * ROWS_PER_TILE, ROWS_PER_TILE)
    pltpu.sync_copy(t_h.at[srows], tsp.at[srows])
    pltpu.sync_copy(src_h.at[pl.ds(base, PER_W)], src_v)
    pltpu.sync_copy(dst_h.at[pl.ds(base, PER_W)], dst_v)
    plsc.subcore_barrier()
    ba = (ba0, ba1)
    bb = (bb0, bb1)
    sga = (sga0, sga1)
    sgb = (sgb0, sgb1)
    swo = (swo0, swo1)

    def start_gather(j, b):
        pltpu.async_copy(tsp.at[src_v.at[pl.ds(j * CH, CH)]], ba[b], sga[b])
        pltpu.async_copy(tsp.at[dst_v.at[pl.ds(j * CH, CH)]], bb[b], sgb[b])

    def wait_gather(b):
        pltpu.make_async_copy(tsp.at[src_v.at[pl.ds(0, CH)]], ba[b], sga[b]).wait()
        pltpu.make_async_copy(tsp.at[dst_v.at[pl.ds(0, CH)]], bb[b], sgb[b]).wait()

    def start_write(j, b):
        pltpu.async_copy(ba[b], out_h.at[pl.ds(base + j * CH, CH)], swo[b])

    def wait_write(b):
        pltpu.make_async_copy(ba[b], out_h.at[pl.ds(base, CH)], swo[b]).wait()

    def merge(b):
        def row(r, carry):
            for k in range(4):
                lanes = pl.ds(2 * HP + 16 * k - H_ + H_, 16) if False else pl.ds(HP + 16 * k, 16)
                ba[b][r, lanes] = bb[b][r, lanes]
            return carry
        lax.fori_loop(0, CH, row, 0)

    start_gather(0, 0)

    def outer(j2, carry):
        for b in range(2):
            j = 2 * j2 + b

            @pl.when(j >= 1)
            def _():
                wait_write(1 - b)

            @pl.when(j + 1 < NCH)
            def _():
                start_gather(j + 1, 1 - b)

            wait_gather(b)
            merge(b)
            start_write(j, b)
        return carry

    lax.fori_loop(0, NCH // 2, outer, 0)
    wait_write(1)


_gather_call = pl.kernel(
    _sc_gather_body,
    out_type=jax.ShapeDtypeStruct((E_PAD, H_), jnp.float32),
    mesh=_SC_MESH,
    scratch_types=[
        pltpu.VMEM((PER_W,), jnp.int32),
        pltpu.VMEM((PER_W,), jnp.int32),
        pltpu.VMEM((CH, H_), jnp.float32),
        pltpu.VMEM((CH, H_), jnp.float32),
        pltpu.VMEM((CH, H_), jnp.float32),
        pltpu.VMEM((CH, H_), jnp.float32),
        pltpu.VMEM_SHARED((N_PAD, H_), jnp.float32),
    ] + [pltpu.SemaphoreType.DMA] * 6,
)


def _sc_scatter_body(ea_h, dst3_h, zeros_h, out_h,
                     idx_v, rb0, rb1, acc, sl0, sl1):
    cid = lax.axis_index("c")
    sid = lax.axis_index("s")
    wid = sid * NC + cid
    base = wid * PER_W
    rows = pl.ds(sid * ROWS_PER_TILE, ROWS_PER_TILE)
    pltpu.sync_copy(zeros_h.at[rows], acc.at[rows])
    pltpu.sync_copy(dst3_h.at[wid], idx_v)
    plsc.subcore_barrier()
    rb = (rb0, rb1)
    sl = (sl0, sl1)

    def start_load(j, b):
        pltpu.async_copy(ea_h.at[pl.ds(base + j * CH, CH)], rb[b], sl[b])

    def wait_load(b):
        pltpu.make_async_copy(ea_h.at[pl.ds(base, CH)], rb[b], sl[b]).wait()

    start_load(0, 0)

    def outer(j2, carry):
        for b in range(2):
            j = 2 * j2 + b

            @pl.when(j + 1 < NCH)
            def _():
                start_load(j + 1, 1 - b)

            wait_load(b)
            pltpu.sync_copy(rb[b], acc.at[idx_v.at[j]], add=True)
        return carry

    lax.fori_loop(0, NCH // 2, outer, 0)
    plsc.subcore_barrier()
    pltpu.sync_copy(acc.at[rows], out_h.at[cid, rows])


_scatter_call = pl.kernel(
    _sc_scatter_body,
    out_type=jax.ShapeDtypeStruct((NC, N_PAD, D_), jnp.float32),
    mesh=_SC_MESH,
    scratch_types=[
        pltpu.VMEM((NCH, CH), jnp.int32),
        pltpu.VMEM((CH, D_), jnp.float32),
        pltpu.VMEM((CH, D_), jnp.float32),
        pltpu.VMEM_SHARED((N_PAD, D_), jnp.float32),
        pltpu.SemaphoreType.DMA,
        pltpu.SemaphoreType.DMA,
    ],
)


TN = 1000
TE = 1024


def _pack_pairs(h):
    u = lax.bitcast_convert_type(h, jnp.uint32)
    r = (u + jnp.uint32(0x7FFF) + ((u >> 16) & jnp.uint32(1))) >> 16
    lo = r[:, :HP]
    hi = r[:, HP:]
    return lax.bitcast_convert_type(lo | (hi << 16), jnp.float32)


def _unpack_pairs(p):
    u = lax.bitcast_convert_type(p, jnp.uint32)
    rep = jnp.concatenate([u, u], axis=1)
    lane = lax.broadcasted_iota(jnp.uint32, rep.shape, 1)
    out_u = jnp.where(lane < HP, rep << 16, rep & jnp.uint32(0xFFFF0000))
    return lax.bitcast_convert_type(out_u, jnp.float32)


def _proj_body(x, w1a, w1b, out):
    xb = x[...]
    out[...] = jnp.concatenate(
        [_pack_pairs(jnp.dot(xb, w1a[...], preferred_element_type=jnp.float32)),
         _pack_pairs(jnp.dot(xb, w1b[...], preferred_element_type=jnp.float32))],
        axis=1)


@jax.jit
def _proj(x, w1a, w1b):
    return pl.pallas_call(
        _proj_body,
        grid=(N_ // TN,),
        in_specs=[
            pl.BlockSpec((TN, D_), lambda i: (i, 0)),
            pl.BlockSpec((D_, H_), lambda i: (0, 0)),
            pl.BlockSpec((D_, H_), lambda i: (0, 0)),
        ],
        out_specs=pl.BlockSpec((TN, H_), lambda i: (i, 0)),
        out_shape=jax.ShapeDtypeStruct((N_, H_), jnp.float32),
    )(x, w1a, w1b)


def _ln_2d(h, g, bt):
    m = jnp.mean(h, axis=-1, keepdims=True)
    v = jnp.mean((h - m) ** 2, axis=-1, keepdims=True)
    return (h - m) * lax.rsqrt(v + 1e-5) * g + bt


def _edge_body(go, ea, w1c, w2, w3, prm, out):
    p = prm[...]
    b1 = p[0:1, :]
    b2 = p[1:2, :]
    b3 = p[2:3, :]
    g = p[3:4, :]
    bt = p[4:5, :]
    eab = ea[...]
    gob = go[...]
    h = (_unpack_pairs(gob[:, :HP]) + _unpack_pairs(gob[:, HP:])
         + jnp.dot(eab, w1c[...], preferred_element_type=jnp.float32) + b1)
    h = jnp.maximum(h, 0.0)
    h = jnp.maximum(jnp.dot(h, w2[...], preferred_element_type=jnp.float32) + b2, 0.0)
    h = jnp.dot(h, w3[...], preferred_element_type=jnp.float32) + b3
    ln = _ln_2d(h, g, bt)
    row = TE * pl.program_id(0) + lax.broadcasted_iota(jnp.int32, (TE, 1), 0)
    out[...] = jnp.where(row < E_, eab + ln, 0.0)


@jax.jit
def _edge(go, ea, w1c, w2, w3, prm):
    return pl.pallas_call(
        _edge_body,
        grid=(E_PAD // TE,),
        in_specs=[
            pl.BlockSpec((TE, H_), lambda i: (i, 0)),
            pl.BlockSpec((TE, D_), lambda i: (i, 0)),
            pl.BlockSpec((D_, H_), lambda i: (0, 0)),
            pl.BlockSpec((H_, H_), lambda i: (0, 0)),
            pl.BlockSpec((H_, D_), lambda i: (0, 0)),
            pl.BlockSpec((8, 128), lambda i: (0, 0)),
        ],
        out_specs=pl.BlockSpec((TE, D_), lambda i: (i, 0)),
        out_shape=jax.ShapeDtypeStruct((E_PAD, D_), jnp.float32),
    )(go, ea, w1c, w2, w3, prm)


def _node_body(x, a0, a1, w1a, w1b, w2, w3, prm, ew1a, ew1b, out, outt):
    p = prm[...]
    b1 = p[0:1, :]
    b2 = p[1:2, :]
    b3 = p[2:3, :]
    g = p[3:4, :]
    bt = p[4:5, :]
    xb = x[...]
    agg = a0[...] + a1[...]
    h = (jnp.dot(xb, w1a[...], preferred_element_type=jnp.float32)
         + jnp.dot(agg, w1b[...], preferred_element_type=jnp.float32) + b1)
    h = jnp.maximum(h, 0.0)
    h = jnp.maximum(jnp.dot(h, w2[...], preferred_element_type=jnp.float32) + b2, 0.0)
    h = jnp.dot(h, w3[...], preferred_element_type=jnp.float32) + b3
    xn = xb + _ln_2d(h, g, bt)
    out[...] = xn
    outt[...] = jnp.concatenate(
        [_pack_pairs(jnp.dot(xn, ew1a[...], preferred_element_type=jnp.float32)),
         _pack_pairs(jnp.dot(xn, ew1b[...], preferred_element_type=jnp.float32))],
        axis=1)


@jax.jit
def _node_proj(x, a0, a1, w1a, w1b, w2, w3, prm, ew1a, ew1b):
    return pl.pallas_call(
        _node_body,
        grid=(N_ // TN,),
        in_specs=[
            pl.BlockSpec((TN, D_), lambda i: (i, 0)),
            pl.BlockSpec((TN, D_), lambda i: (i, 0)),
            pl.BlockSpec((TN, D_), lambda i: (i, 0)),
            pl.BlockSpec((D_, H_), lambda i: (0, 0)),
            pl.BlockSpec((D_, H_), lambda i: (0, 0)),
            pl.BlockSpec((H_, H_), lambda i: (0, 0)),
            pl.BlockSpec((H_, D_), lambda i: (0, 0)),
            pl.BlockSpec((8, 128), lambda i: (0, 0)),
            pl.BlockSpec((D_, H_), lambda i: (0, 0)),
            pl.BlockSpec((D_, H_), lambda i: (0, 0)),
        ],
        out_specs=[pl.BlockSpec((TN, D_), lambda i: (i, 0)),
                   pl.BlockSpec((TN, H_), lambda i: (i, 0))],
        out_shape=[jax.ShapeDtypeStruct((N_, D_), jnp.float32),
                   jax.ShapeDtypeStruct((N_, H_), jnp.float32)],
    )(x, a0, a1, w1a, w1b, w2, w3, prm, ew1a, ew1b)


def _node_last_body(x, a0, a1, w1a, w1b, w2, w3, prm, out):
    p = prm[...]
    b1 = p[0:1, :]
    b2 = p[1:2, :]
    b3 = p[2:3, :]
    g = p[3:4, :]
    bt = p[4:5, :]
    xb = x[...]
    agg = a0[...] + a1[...]
    h = (jnp.dot(xb, w1a[...], preferred_element_type=jnp.float32)
         + jnp.dot(agg, w1b[...], preferred_element_type=jnp.float32) + b1)
    h = jnp.maximum(h, 0.0)
    h = jnp.maximum(jnp.dot(h, w2[...], preferred_element_type=jnp.float32) + b2, 0.0)
    h = jnp.dot(h, w3[...], preferred_element_type=jnp.float32) + b3
    out[...] = xb + _ln_2d(h, g, bt)


@jax.jit
def _node_last(x, a0, a1, w1a, w1b, w2, w3, prm):
    return pl.pallas_call(
        _node_last_body,
        grid=(N_ // TN,),
        in_specs=[
            pl.BlockSpec((TN, D_), lambda i: (i, 0)),
            pl.BlockSpec((TN, D_), lambda i: (i, 0)),
            pl.BlockSpec((TN, D_), lambda i: (i, 0)),
            pl.BlockSpec((D_, H_), lambda i: (0, 0)),
            pl.BlockSpec((D_, H_), lambda i: (0, 0)),
            pl.BlockSpec((H_, H_), lambda i: (0, 0)),
            pl.BlockSpec((H_, D_), lambda i: (0, 0)),
            pl.BlockSpec((8, 128), lambda i: (0, 0)),
        ],
        out_specs=pl.BlockSpec((TN, D_), lambda i: (i, 0)),
        out_shape=jax.ShapeDtypeStruct((N_, D_), jnp.float32),
    )(x, a0, a1, w1a, w1b, w2, w3, prm)


def _pack_params(b1, b2, b3, g, bt):
    p = jnp.stack([b1, b2, b3, g, bt], axis=1)
    return jnp.pad(p, ((0, 0), (0, 3), (0, 0)))


def kernel(x, edge_index, edge_attr, eW1, eb1, eW2, eb2, eW3, eb3, eg, ebt,
           nW1, nb1, nW2, nb2, nW3, nb3, ng, nbt):
    pad = E_PAD - E_
    src_p = jnp.pad(edge_index[0], (0, pad))
    dst_p = jnp.pad(edge_index[1], (0, pad))
    dst3 = dst_p.reshape(NW, NCH, CH)
    ea = jnp.pad(edge_attr, ((0, pad), (0, 0)))
    zeros_nd = jnp.zeros((N_PAD, D_), jnp.float32)

    eW1a = eW1[:, :D_, :]
    eW1b = eW1[:, D_:2 * D_, :]
    eW1c = eW1[:, 2 * D_:, :]
    nW1a = nW1[:, :D_, :]
    nW1b = nW1[:, D_:, :]
    eprm = _pack_params(eb1, eb2, eb3, eg, ebt)
    nprm = _pack_params(nb1, nb2, nb3, ng, nbt)

    tbl = jnp.pad(_proj(x, eW1a[0], eW1b[0]), ((0, N_PAD - N_), (0, 0)))
    for i in range(NBLK):
        go = _gather_call(tbl, src_p, dst_p)
        ea = _edge(go, ea, eW1c[i], eW2[i], eW3[i], eprm[i])
        parts = _scatter_call(ea, dst3, zeros_nd)
        if i + 1 < NBLK:
            x, tbl = _node_proj(x, parts[0, :N_], parts[1, :N_],
                                nW1a[i], nW1b[i], nW2[i], nW3[i], nprm[i],
                                eW1a[i + 1], eW1b[i + 1])
            tbl = jnp.pad(tbl, ((0, N_PAD - N_), (0, 0)))
        else:
            x = _node_last(x, parts[0, :N_], parts[1, :N_],
                           nW1a[i], nW1b[i], nW2[i], nW3[i], nprm[i])
    return x


# speedup vs baseline: 3.8888x; 1.0245x over previous
import functools

import jax
import jax.numpy as jnp
from jax import lax
from jax.experimental import pallas as pl
from jax.experimental.pallas import tpu as pltpu
from jax.experimental.pallas import tpu_sc as plsc

N_ = 10000
E_ = 160000
D_ = 128
H_ = 128
HP = H_ // 2
NBLK = 9

NC = 2
NS = 16
NW = NC * NS

CH = 64
PER_W = 5120
E_PAD = NW * PER_W
NCH = PER_W // CH
CHS = 128
NCHS = PER_W // CHS
N_PAD = 10240
ROWS_PER_TILE = N_PAD // NS

_SC_MESH = plsc.VectorSubcoreMesh(
    core_axis_name="c", subcore_axis_name="s", num_cores=NC, num_subcores=NS)


def _sc_gather_body(t_h, src_h, dst_h, out_h,
                    src_v, dst_v, ba0, ba1, bb0, bb1, tsp,
                    sga0, sga1, sgb0, sgb1, swo0, swo1):
    cid = lax.axis_index("c")
    sid = lax.axis_index("s")
    wid = sid * NC + cid
    base = wid * PER_W
    srows = pl.ds(sid * ROWS_PER_TILE, ROWS_PER_TILE)
    pltpu.sync_copy(t_h.at[srows], tsp.at[srows])
    pltpu.sync_copy(src_h.at[pl.ds(base, PER_W)], src_v)
    pltpu.sync_copy(dst_h.at[pl.ds(base, PER_W)], dst_v)
    plsc.subcore_barrier()
    ba = (ba0, ba1)
    bb = (bb0, bb1)
    sga = (sga0, sga1)
    sgb = (sgb0, sgb1)
    swo = (swo0, swo1)

    def start_gather(j, b):
        pltpu.async_copy(tsp.at[src_v.at[pl.ds(j * CH, CH)]], ba[b], sga[b])
        pltpu.async_copy(tsp.at[dst_v.at[pl.ds(j * CH, CH)]], bb[b], sgb[b])

    def wait_gather(b):
        pltpu.make_async_copy(tsp.at[src_v.at[pl.ds(0, CH)]], ba[b], sga[b]).wait()
        pltpu.make_async_copy(tsp.at[dst_v.at[pl.ds(0, CH)]], bb[b], sgb[b]).wait()

    def start_write(j, b):
        pltpu.async_copy(ba[b], out_h.at[pl.ds(base + j * CH, CH)], swo[b])

    def wait_write(b):
        pltpu.make_async_copy(ba[b], out_h.at[pl.ds(base, CH)], swo[b]).wait()

    def merge(b):
        def row(r, carry):
            for k in range(4):
                lanes = pl.ds(2 * HP + 16 * k - H_ + H_, 16) if False else pl.ds(HP + 16 * k, 16)
                ba[b][r, lanes] = bb[b][r, lanes]
            return carry
        lax.fori_loop(0, CH, row, 0)

    start_gather(0, 0)

    def outer(j2, carry):
        for b in range(2):
            j = 2 * j2 + b

            @pl.when(j >= 1)
            def _():
                wait_write(1 - b)

            @pl.when(j + 1 < NCH)
            def _():
                start_gather(j + 1, 1 - b)

            wait_gather(b)
            merge(b)
            start_write(j, b)
        return carry

    lax.fori_loop(0, NCH // 2, outer, 0)
    wait_write(1)


_gather_call = pl.kernel(
    _sc_gather_body,
    out_type=jax.ShapeDtypeStruct((E_PAD, H_), jnp.float32),
    mesh=_SC_MESH,
    scratch_types=[
        pltpu.VMEM((PER_W,), jnp.int32),
        pltpu.VMEM((PER_W,), jnp.int32),
        pltpu.VMEM((CH, H_), jnp.float32),
        pltpu.VMEM((CH, H_), jnp.float32),
        pltpu.VMEM((CH, H_), jnp.float32),
        pltpu.VMEM((CH, H_), jnp.float32),
        pltpu.VMEM_SHARED((N_PAD, H_), jnp.float32),
    ] + [pltpu.SemaphoreType.DMA] * 6,
)


def _sc_scatter_body(ea_h, dst3_h, zeros_h, out_h,
                     idx_v, rb0, rb1, acc, sl0, sl1, sa0, sa1):
    cid = lax.axis_index("c")
    sid = lax.axis_index("s")
    wid = sid * NC + cid
    base = wid * PER_W
    rows = pl.ds(sid * ROWS_PER_TILE, ROWS_PER_TILE)
    pltpu.sync_copy(zeros_h.at[rows], acc.at[rows])
    pltpu.sync_copy(dst3_h.at[wid], idx_v)
    plsc.subcore_barrier()
    rb = (rb0, rb1)
    sl = (sl0, sl1)
    sa = (sa0, sa1)

    def start_load(j, b):
        pltpu.async_copy(ea_h.at[pl.ds(base + j * CHS, CHS)], rb[b], sl[b])

    def wait_load(b):
        pltpu.make_async_copy(ea_h.at[pl.ds(base, CHS)], rb[b], sl[b]).wait()

    def start_add(j, b):
        pltpu.async_copy(rb[b], acc.at[idx_v.at[j]], sa[b], add=True)

    def wait_add(b):
        pltpu.make_async_copy(rb[b], acc.at[idx_v.at[0]], sa[b]).wait()

    start_load(0, 0)

    def outer(j2, carry):
        for b in range(2):
            j = 2 * j2 + b

            @pl.when(j >= 2)
            def _():
                wait_add(b)

            @pl.when(j + 1 < NCHS)
            def _():
                start_load(j + 1, 1 - b)

            wait_load(b)
            start_add(j, b)
        return carry

    lax.fori_loop(0, NCHS // 2, outer, 0)
    wait_add(0)
    wait_add(1)
    plsc.subcore_barrier()
    pltpu.sync_copy(acc.at[rows], out_h.at[cid, rows])


_scatter_call = pl.kernel(
    _sc_scatter_body,
    out_type=jax.ShapeDtypeStruct((NC, N_PAD, D_), jnp.float32),
    mesh=_SC_MESH,
    scratch_types=[
        pltpu.VMEM((NCHS, CHS), jnp.int32),
        pltpu.VMEM((CHS, D_), jnp.float32),
        pltpu.VMEM((CHS, D_), jnp.float32),
        pltpu.VMEM_SHARED((N_PAD, D_), jnp.float32),
        pltpu.SemaphoreType.DMA,
        pltpu.SemaphoreType.DMA,
        pltpu.SemaphoreType.DMA,
        pltpu.SemaphoreType.DMA,
    ],
)


TN = 1000
TE = 1024


def _pack_pairs(h):
    u = lax.bitcast_convert_type(h, jnp.uint32)
    r = (u + jnp.uint32(0x7FFF) + ((u >> 16) & jnp.uint32(1))) >> 16
    lo = r[:, :HP]
    hi = r[:, HP:]
    return lax.bitcast_convert_type(lo | (hi << 16), jnp.float32)


def _unpack_pairs(p):
    u = lax.bitcast_convert_type(p, jnp.uint32)
    rep = jnp.concatenate([u, u], axis=1)
    lane = lax.broadcasted_iota(jnp.uint32, rep.shape, 1)
    out_u = jnp.where(lane < HP, rep << 16, rep & jnp.uint32(0xFFFF0000))
    return lax.bitcast_convert_type(out_u, jnp.float32)


def _proj_body(x, w1a, w1b, out):
    xb = x[...]
    out[...] = jnp.concatenate(
        [_pack_pairs(jnp.dot(xb, w1a[...], preferred_element_type=jnp.float32)),
         _pack_pairs(jnp.dot(xb, w1b[...], preferred_element_type=jnp.float32))],
        axis=1)


@jax.jit
def _proj(x, w1a, w1b):
    return pl.pallas_call(
        _proj_body,
        grid=(N_ // TN,),
        in_specs=[
            pl.BlockSpec((TN, D_), lambda i: (i, 0)),
            pl.BlockSpec((D_, H_), lambda i: (0, 0)),
            pl.BlockSpec((D_, H_), lambda i: (0, 0)),
        ],
        out_specs=pl.BlockSpec((TN, H_), lambda i: (i, 0)),
        out_shape=jax.ShapeDtypeStruct((N_, H_), jnp.float32),
    )(x, w1a, w1b)


def _ln_2d(h, g, bt):
    m = jnp.mean(h, axis=-1, keepdims=True)
    v = jnp.mean((h - m) ** 2, axis=-1, keepdims=True)
    return (h - m) * lax.rsqrt(v + 1e-5) * g + bt


def _edge_body(go, ea, w1c, w2, w3, prm, out):
    p = prm[...]
    b1 = p[0:1, :]
    b2 = p[1:2, :]
    b3 = p[2:3, :]
    g = p[3:4, :]
    bt = p[4:5, :]
    eab = ea[...]
    gob = go[...]
    h = (_unpack_pairs(gob[:, :HP]) + _unpack_pairs(gob[:, HP:])
         + jnp.dot(eab, w1c[...], preferred_element_type=jnp.float32) + b1)
    h = jnp.maximum(h, 0.0)
    h = jnp.maximum(jnp.dot(h, w2[...], preferred_element_type=jnp.float32) + b2, 0.0)
    h = jnp.dot(h, w3[...], preferred_element_type=jnp.float32) + b3
    ln = _ln_2d(h, g, bt)
    row = TE * pl.program_id(0) + lax.broadcasted_iota(jnp.int32, (TE, 1), 0)
    out[...] = jnp.where(row < E_, eab + ln, 0.0)


@jax.jit
def _edge(go, ea, w1c, w2, w3, prm):
    return pl.pallas_call(
        _edge_body,
        grid=(E_PAD // TE,),
        in_specs=[
            pl.BlockSpec((TE, H_), lambda i: (i, 0)),
            pl.BlockSpec((TE, D_), lambda i: (i, 0)),
            pl.BlockSpec((D_, H_), lambda i: (0, 0)),
            pl.BlockSpec((H_, H_), lambda i: (0, 0)),
            pl.BlockSpec((H_, D_), lambda i: (0, 0)),
            pl.BlockSpec((8, 128), lambda i: (0, 0)),
        ],
        out_specs=pl.BlockSpec((TE, D_), lambda i: (i, 0)),
        out_shape=jax.ShapeDtypeStruct((E_PAD, D_), jnp.float32),
    )(go, ea, w1c, w2, w3, prm)


def _node_body(x, a0, a1, w1a, w1b, w2, w3, prm, ew1a, ew1b, out, outt):
    p = prm[...]
    b1 = p[0:1, :]
    b2 = p[1:2, :]
    b3 = p[2:3, :]
    g = p[3:4, :]
    bt = p[4:5, :]
    xb = x[...]
    agg = a0[...] + a1[...]
    h = (jnp.dot(xb, w1a[...], preferred_element_type=jnp.float32)
         + jnp.dot(agg, w1b[...], preferred_element_type=jnp.float32) + b1)
    h = jnp.maximum(h, 0.0)
    h = jnp.maximum(jnp.dot(h, w2[...], preferred_element_type=jnp.float32) + b2, 0.0)
    h = jnp.dot(h, w3[...], preferred_element_type=jnp.float32) + b3
    xn = xb + _ln_2d(h, g, bt)
    out[...] = xn
    outt[...] = jnp.concatenate(
        [_pack_pairs(jnp.dot(xn, ew1a[...], preferred_element_type=jnp.float32)),
         _pack_pairs(jnp.dot(xn, ew1b[...], preferred_element_type=jnp.float32))],
        axis=1)


@jax.jit
def _node_proj(x, a0, a1, w1a, w1b, w2, w3, prm, ew1a, ew1b):
    return pl.pallas_call(
        _node_body,
        grid=(N_ // TN,),
        in_specs=[
            pl.BlockSpec((TN, D_), lambda i: (i, 0)),
            pl.BlockSpec((TN, D_), lambda i: (i, 0)),
            pl.BlockSpec((TN, D_), lambda i: (i, 0)),
            pl.BlockSpec((D_, H_), lambda i: (0, 0)),
            pl.BlockSpec((D_, H_), lambda i: (0, 0)),
            pl.BlockSpec((H_, H_), lambda i: (0, 0)),
            pl.BlockSpec((H_, D_), lambda i: (0, 0)),
            pl.BlockSpec((8, 128), lambda i: (0, 0)),
            pl.BlockSpec((D_, H_), lambda i: (0, 0)),
            pl.BlockSpec((D_, H_), lambda i: (0, 0)),
        ],
        out_specs=[pl.BlockSpec((TN, D_), lambda i: (i, 0)),
                   pl.BlockSpec((TN, H_), lambda i: (i, 0))],
        out_shape=[jax.ShapeDtypeStruct((N_, D_), jnp.float32),
                   jax.ShapeDtypeStruct((N_, H_), jnp.float32)],
    )(x, a0, a1, w1a, w1b, w2, w3, prm, ew1a, ew1b)


def _node_last_body(x, a0, a1, w1a, w1b, w2, w3, prm, out):
    p = prm[...]
    b1 = p[0:1, :]
    b2 = p[1:2, :]
    b3 = p[2:3, :]
    g = p[3:4, :]
    bt = p[4:5, :]
    xb = x[...]
    agg = a0[...] + a1[...]
    h = (jnp.dot(xb, w1a[...], preferred_element_type=jnp.float32)
         + jnp.dot(agg, w1b[...], preferred_element_type=jnp.float32) + b1)
    h = jnp.maximum(h, 0.0)
    h = jnp.maximum(jnp.dot(h, w2[...], preferred_element_type=jnp.float32) + b2, 0.0)
    h = jnp.dot(h, w3[...], preferred_element_type=jnp.float32) + b3
    out[...] = xb + _ln_2d(h, g, bt)


@jax.jit
def _node_last(x, a0, a1, w1a, w1b, w2, w3, prm):
    return pl.pallas_call(
        _node_last_body,
        grid=(N_ // TN,),
        in_specs=[
            pl.BlockSpec((TN, D_), lambda i: (i, 0)),
            pl.BlockSpec((TN, D_), lambda i: (i, 0)),
            pl.BlockSpec((TN, D_), lambda i: (i, 0)),
            pl.BlockSpec((D_, H_), lambda i: (0, 0)),
            pl.BlockSpec((D_, H_), lambda i: (0, 0)),
            pl.BlockSpec((H_, H_), lambda i: (0, 0)),
            pl.BlockSpec((H_, D_), lambda i: (0, 0)),
            pl.BlockSpec((8, 128), lambda i: (0, 0)),
        ],
        out_specs=pl.BlockSpec((TN, D_), lambda i: (i, 0)),
        out_shape=jax.ShapeDtypeStruct((N_, D_), jnp.float32),
    )(x, a0, a1, w1a, w1b, w2, w3, prm)


def _pack_params(b1, b2, b3, g, bt):
    p = jnp.stack([b1, b2, b3, g, bt], axis=1)
    return jnp.pad(p, ((0, 0), (0, 3), (0, 0)))


def kernel(x, edge_index, edge_attr, eW1, eb1, eW2, eb2, eW3, eb3, eg, ebt,
           nW1, nb1, nW2, nb2, nW3, nb3, ng, nbt):
    pad = E_PAD - E_
    src_p = jnp.pad(edge_index[0], (0, pad))
    dst_p = jnp.pad(edge_index[1], (0, pad))
    dst3 = dst_p.reshape(NW, NCHS, CHS)
    ea = jnp.pad(edge_attr, ((0, pad), (0, 0)))
    zeros_nd = jnp.zeros((N_PAD, D_), jnp.float32)

    eW1a = eW1[:, :D_, :]
    eW1b = eW1[:, D_:2 * D_, :]
    eW1c = eW1[:, 2 * D_:, :]
    nW1a = nW1[:, :D_, :]
    nW1b = nW1[:, D_:, :]
    eprm = _pack_params(eb1, eb2, eb3, eg, ebt)
    nprm = _pack_params(nb1, nb2, nb3, ng, nbt)

    tbl = jnp.pad(_proj(x, eW1a[0], eW1b[0]), ((0, N_PAD - N_), (0, 0)))
    for i in range(NBLK):
        go = _gather_call(tbl, src_p, dst_p)
        ea = _edge(go, ea, eW1c[i], eW2[i], eW3[i], eprm[i])
        parts = _scatter_call(ea, dst3, zeros_nd)
        if i + 1 < NBLK:
            x, tbl = _node_proj(x, parts[0, :N_], parts[1, :N_],
                                nW1a[i], nW1b[i], nW2[i], nW3[i], nprm[i],
                                eW1a[i + 1], eW1b[i + 1])
            tbl = jnp.pad(tbl, ((0, N_PAD - N_), (0, 0)))
        else:
            x = _node_last(x, parts[0, :N_], parts[1, :N_],
                           nW1a[i], nW1b[i], nW2[i], nW3[i], nprm[i])
    return x


# speedup vs baseline: 3.9112x; 1.0058x over previous
import functools

import jax
import jax.numpy as jnp
from jax import lax
from jax.experimental import pallas as pl
from jax.experimental.pallas import tpu as pltpu
from jax.experimental.pallas import tpu_sc as plsc

N_ = 10000
E_ = 160000
D_ = 128
H_ = 128
HP = H_ // 2
NBLK = 9

NC = 2
NS = 16
NW = NC * NS

CH = 64
PER_W = 5120
E_PAD = NW * PER_W
NCH = PER_W // CH
CHS = 128
NCHS = PER_W // CHS
N_PAD = 10240
ROWS_PER_TILE = N_PAD // NS

_SC_MESH = plsc.VectorSubcoreMesh(
    core_axis_name="c", subcore_axis_name="s", num_cores=NC, num_subcores=NS)


def _sc_gather_body(t_h, src_h, dst_h, out_h,
                    src_v, dst_v, ba0, ba1, bb0, bb1, tsp,
                    sga0, sga1, sgb0, sgb1, swo0, swo1):
    cid = lax.axis_index("c")
    sid = lax.axis_index("s")
    wid = sid * NC + cid
    base = wid * PER_W
    srows = pl.ds(sid * ROWS_PER_TILE, ROWS_PER_TILE)
    pltpu.sync_copy(t_h.at[srows], tsp.at[srows])
    pltpu.sync_copy(src_h.at[pl.ds(base, PER_W)], src_v)
    pltpu.sync_copy(dst_h.at[pl.ds(base, PER_W)], dst_v)
    plsc.subcore_barrier()
    ba = (ba0, ba1)
    bb = (bb0, bb1)
    sga = (sga0, sga1)
    sgb = (sgb0, sgb1)
    swo = (swo0, swo1)

    def start_gather(j, b):
        pltpu.async_copy(tsp.at[src_v.at[pl.ds(j * CH, CH)]], ba[b], sga[b])
        pltpu.async_copy(tsp.at[dst_v.at[pl.ds(j * CH, CH)]], bb[b], sgb[b])

    def wait_gather(b):
        pltpu.make_async_copy(tsp.at[src_v.at[pl.ds(0, CH)]], ba[b], sga[b]).wait()
        pltpu.make_async_copy(tsp.at[dst_v.at[pl.ds(0, CH)]], bb[b], sgb[b]).wait()

    def start_write(j, b):
        pltpu.async_copy(ba[b], out_h.at[pl.ds(base + j * CH, CH)], swo[b])

    def wait_write(b):
        pltpu.make_async_copy(ba[b], out_h.at[pl.ds(base, CH)], swo[b]).wait()

    def merge(b):
        def row(r, carry):
            for k in range(4):
                lanes = pl.ds(2 * HP + 16 * k - H_ + H_, 16) if False else pl.ds(HP + 16 * k, 16)
                ba[b][r, lanes] = bb[b][r, lanes]
            return carry
        lax.fori_loop(0, CH, row, 0)

    start_gather(0, 0)

    def outer(j2, carry):
        for b in range(2):
            j = 2 * j2 + b

            @pl.when(j >= 1)
            def _():
                wait_write(1 - b)

            @pl.when(j + 1 < NCH)
            def _():
                start_gather(j + 1, 1 - b)

            wait_gather(b)
            merge(b)
            start_write(j, b)
        return carry

    lax.fori_loop(0, NCH // 2, outer, 0)
    wait_write(1)


_gather_call = pl.kernel(
    _sc_gather_body,
    out_type=jax.ShapeDtypeStruct((E_PAD, H_), jnp.float32),
    mesh=_SC_MESH,
    scratch_types=[
        pltpu.VMEM((PER_W,), jnp.int32),
        pltpu.VMEM((PER_W,), jnp.int32),
        pltpu.VMEM((CH, H_), jnp.float32),
        pltpu.VMEM((CH, H_), jnp.float32),
        pltpu.VMEM((CH, H_), jnp.float32),
        pltpu.VMEM((CH, H_), jnp.float32),
        pltpu.VMEM_SHARED((N_PAD, H_), jnp.float32),
    ] + [pltpu.SemaphoreType.DMA] * 6,
)


def _sc_scatter_body(ea_h, dst3_h, zeros_h, out_h,
                     idx_v, rb0, rb1, acc, sl0, sl1, sa0, sa1):
    cid = lax.axis_index("c")
    sid = lax.axis_index("s")
    wid = sid * NC + cid
    base = wid * PER_W
    rows = pl.ds(sid * ROWS_PER_TILE, ROWS_PER_TILE)
    pltpu.sync_copy(zeros_h.at[rows], acc.at[rows])
    pltpu.sync_copy(dst3_h.at[wid], idx_v)
    plsc.subcore_barrier()
    rb = (rb0, rb1)
    sl = (sl0, sl1)
    sa = (sa0, sa1)

    def start_load(j, b):
        pltpu.async_copy(ea_h.at[pl.ds(base + j * CHS, CHS)], rb[b], sl[b])

    def wait_load(b):
        pltpu.make_async_copy(ea_h.at[pl.ds(base, CHS)], rb[b], sl[b]).wait()

    def start_add(j, b):
        pltpu.async_copy(rb[b], acc.at[idx_v.at[j]], sa[b], add=True)

    def wait_add(b):
        pltpu.make_async_copy(rb[b], acc.at[idx_v.at[0]], sa[b]).wait()

    start_load(0, 0)

    def outer(j2, carry):
        for b in range(2):
            j = 2 * j2 + b

            @pl.when(j >= 1)
            def _():
                wait_add(1 - b)

            @pl.when(j + 1 < NCHS)
            def _():
                start_load(j + 1, 1 - b)

            wait_load(b)
            start_add(j, b)
        return carry

    lax.fori_loop(0, NCHS // 2, outer, 0)
    wait_add(1)
    plsc.subcore_barrier()
    pltpu.sync_copy(acc.at[rows], out_h.at[cid, rows])


_scatter_call = pl.kernel(
    _sc_scatter_body,
    out_type=jax.ShapeDtypeStruct((NC, N_PAD, D_), jnp.float32),
    mesh=_SC_MESH,
    scratch_types=[
        pltpu.VMEM((NCHS, CHS), jnp.int32),
        pltpu.VMEM((CHS, D_), jnp.float32),
        pltpu.VMEM((CHS, D_), jnp.float32),
        pltpu.VMEM_SHARED((N_PAD, D_), jnp.float32),
        pltpu.SemaphoreType.DMA,
        pltpu.SemaphoreType.DMA,
        pltpu.SemaphoreType.DMA,
        pltpu.SemaphoreType.DMA,
    ],
)


TN = 1000
TE = 1024


def _pack_pairs(h):
    u = lax.bitcast_convert_type(h, jnp.uint32)
    r = (u + jnp.uint32(0x7FFF) + ((u >> 16) & jnp.uint32(1))) >> 16
    lo = r[:, :HP]
    hi = r[:, HP:]
    return lax.bitcast_convert_type(lo | (hi << 16), jnp.float32)


def _unpack_pairs(p):
    u = lax.bitcast_convert_type(p, jnp.uint32)
    rep = jnp.concatenate([u, u], axis=1)
    lane = lax.broadcasted_iota(jnp.uint32, rep.shape, 1)
    out_u = jnp.where(lane < HP, rep << 16, rep & jnp.uint32(0xFFFF0000))
    return lax.bitcast_convert_type(out_u, jnp.float32)


def _proj_body(x, w1a, w1b, out):
    xb = x[...]
    out[...] = jnp.concatenate(
        [_pack_pairs(jnp.dot(xb, w1a[...], preferred_element_type=jnp.float32)),
         _pack_pairs(jnp.dot(xb, w1b[...], preferred_element_type=jnp.float32))],
        axis=1)


@jax.jit
def _proj(x, w1a, w1b):
    return pl.pallas_call(
        _proj_body,
        grid=(N_ // TN,),
        in_specs=[
            pl.BlockSpec((TN, D_), lambda i: (i, 0)),
            pl.BlockSpec((D_, H_), lambda i: (0, 0)),
            pl.BlockSpec((D_, H_), lambda i: (0, 0)),
        ],
        out_specs=pl.BlockSpec((TN, H_), lambda i: (i, 0)),
        out_shape=jax.ShapeDtypeStruct((N_, H_), jnp.float32),
    )(x, w1a, w1b)


def _ln_2d(h, g, bt):
    m = jnp.mean(h, axis=-1, keepdims=True)
    v = jnp.mean((h - m) ** 2, axis=-1, keepdims=True)
    return (h - m) * lax.rsqrt(v + 1e-5) * g + bt


def _edge_body(go, ea, w1c, w2, w3, prm, out):
    p = prm[...]
    b1 = p[0:1, :]
    b2 = p[1:2, :]
    b3 = p[2:3, :]
    g = p[3:4, :]
    bt = p[4:5, :]
    eab = ea[...]
    gob = go[...]
    h = (_unpack_pairs(gob[:, :HP]) + _unpack_pairs(gob[:, HP:])
         + jnp.dot(eab, w1c[...], preferred_element_type=jnp.float32) + b1)
    h = jnp.maximum(h, 0.0)
    h = jnp.maximum(jnp.dot(h, w2[...], preferred_element_type=jnp.float32) + b2, 0.0)
    h = jnp.dot(h, w3[...], preferred_element_type=jnp.float32) + b3
    ln = _ln_2d(h, g, bt)
    row = TE * pl.program_id(0) + lax.broadcasted_iota(jnp.int32, (TE, 1), 0)
    out[...] = jnp.where(row < E_, eab + ln, 0.0)


@jax.jit
def _edge(go, ea, w1c, w2, w3, prm):
    return pl.pallas_call(
        _edge_body,
        grid=(E_PAD // TE,),
        in_specs=[
            pl.BlockSpec((TE, H_), lambda i: (i, 0)),
            pl.BlockSpec((TE, D_), lambda i: (i, 0)),
            pl.BlockSpec((D_, H_), lambda i: (0, 0)),
            pl.BlockSpec((H_, H_), lambda i: (0, 0)),
            pl.BlockSpec((H_, D_), lambda i: (0, 0)),
            pl.BlockSpec((8, 128), lambda i: (0, 0)),
        ],
        out_specs=pl.BlockSpec((TE, D_), lambda i: (i, 0)),
        out_shape=jax.ShapeDtypeStruct((E_PAD, D_), jnp.float32),
    )(go, ea, w1c, w2, w3, prm)


def _node_body(x, a0, a1, w1a, w1b, w2, w3, prm, ew1a, ew1b, out, outt):
    p = prm[...]
    b1 = p[0:1, :]
    b2 = p[1:2, :]
    b3 = p[2:3, :]
    g = p[3:4, :]
    bt = p[4:5, :]
    xb = x[...]
    agg = a0[...] + a1[...]
    h = (jnp.dot(xb, w1a[...], preferred_element_type=jnp.float32)
         + jnp.dot(agg, w1b[...], preferred_element_type=jnp.float32) + b1)
    h = jnp.maximum(h, 0.0)
    h = jnp.maximum(jnp.dot(h, w2[...], preferred_element_type=jnp.float32) + b2, 0.0)
    h = jnp.dot(h, w3[...], preferred_element_type=jnp.float32) + b3
    xn = xb + _ln_2d(h, g, bt)
    out[...] = xn
    outt[...] = jnp.concatenate(
        [_pack_pairs(jnp.dot(xn, ew1a[...], preferred_element_type=jnp.float32)),
         _pack_pairs(jnp.dot(xn, ew1b[...], preferred_element_type=jnp.float32))],
        axis=1)


@jax.jit
def _node_proj(x, a0, a1, w1a, w1b, w2, w3, prm, ew1a, ew1b):
    return pl.pallas_call(
        _node_body,
        grid=(N_ // TN,),
        in_specs=[
            pl.BlockSpec((TN, D_), lambda i: (i, 0)),
            pl.BlockSpec((TN, D_), lambda i: (i, 0)),
            pl.BlockSpec((TN, D_), lambda i: (i, 0)),
            pl.BlockSpec((D_, H_), lambda i: (0, 0)),
            pl.BlockSpec((D_, H_), lambda i: (0, 0)),
            pl.BlockSpec((H_, H_), lambda i: (0, 0)),
            pl.BlockSpec((H_, D_), lambda i: (0, 0)),
            pl.BlockSpec((8, 128), lambda i: (0, 0)),
            pl.BlockSpec((D_, H_), lambda i: (0, 0)),
            pl.BlockSpec((D_, H_), lambda i: (0, 0)),
        ],
        out_specs=[pl.BlockSpec((TN, D_), lambda i: (i, 0)),
                   pl.BlockSpec((TN, H_), lambda i: (i, 0))],
        out_shape=[jax.ShapeDtypeStruct((N_, D_), jnp.float32),
                   jax.ShapeDtypeStruct((N_, H_), jnp.float32)],
    )(x, a0, a1, w1a, w1b, w2, w3, prm, ew1a, ew1b)


def _node_last_body(x, a0, a1, w1a, w1b, w2, w3, prm, out):
    p = prm[...]
    b1 = p[0:1, :]
    b2 = p[1:2, :]
    b3 = p[2:3, :]
    g = p[3:4, :]
    bt = p[4:5, :]
    xb = x[...]
    agg = a0[...] + a1[...]
    h = (jnp.dot(xb, w1a[...], preferred_element_type=jnp.float32)
         + jnp.dot(agg, w1b[...], preferred_element_type=jnp.float32) + b1)
    h = jnp.maximum(h, 0.0)
    h = jnp.maximum(jnp.dot(h, w2[...], preferred_element_type=jnp.float32) + b2, 0.0)
    h = jnp.dot(h, w3[...], preferred_element_type=jnp.float32) + b3
    out[...] = xb + _ln_2d(h, g, bt)


@jax.jit
def _node_last(x, a0, a1, w1a, w1b, w2, w3, prm):
    return pl.pallas_call(
        _node_last_body,
        grid=(N_ // TN,),
        in_specs=[
            pl.BlockSpec((TN, D_), lambda i: (i, 0)),
            pl.BlockSpec((TN, D_), lambda i: (i, 0)),
            pl.BlockSpec((TN, D_), lambda i: (i, 0)),
            pl.BlockSpec((D_, H_), lambda i: (0, 0)),
            pl.BlockSpec((D_, H_), lambda i: (0, 0)),
            pl.BlockSpec((H_, H_), lambda i: (0, 0)),
            pl.BlockSpec((H_, D_), lambda i: (0, 0)),
            pl.BlockSpec((8, 128), lambda i: (0, 0)),
        ],
        out_specs=pl.BlockSpec((TN, D_), lambda i: (i, 0)),
        out_shape=jax.ShapeDtypeStruct((N_, D_), jnp.float32),
    )(x, a0, a1, w1a, w1b, w2, w3, prm)


def _pack_params(b1, b2, b3, g, bt):
    p = jnp.stack([b1, b2, b3, g, bt], axis=1)
    return jnp.pad(p, ((0, 0), (0, 3), (0, 0)))


def kernel(x, edge_index, edge_attr, eW1, eb1, eW2, eb2, eW3, eb3, eg, ebt,
           nW1, nb1, nW2, nb2, nW3, nb3, ng, nbt):
    pad = E_PAD - E_
    src_p = jnp.pad(edge_index[0], (0, pad))
    dst_p = jnp.pad(edge_index[1], (0, pad))
    dst3 = dst_p.reshape(NW, NCHS, CHS)
    ea = jnp.pad(edge_attr, ((0, pad), (0, 0)))
    zeros_nd = jnp.zeros((N_PAD, D_), jnp.float32)

    eW1a = eW1[:, :D_, :]
    eW1b = eW1[:, D_:2 * D_, :]
    eW1c = eW1[:, 2 * D_:, :]
    nW1a = nW1[:, :D_, :]
    nW1b = nW1[:, D_:, :]
    eprm = _pack_params(eb1, eb2, eb3, eg, ebt)
    nprm = _pack_params(nb1, nb2, nb3, ng, nbt)

    tbl = jnp.pad(_proj(x, eW1a[0], eW1b[0]), ((0, N_PAD - N_), (0, 0)))
    for i in range(NBLK):
        go = _gather_call(tbl, src_p, dst_p)
        ea = _edge(go, ea, eW1c[i], eW2[i], eW3[i], eprm[i])
        parts = _scatter_call(ea, dst3, zeros_nd)
        if i + 1 < NBLK:
            x, tbl = _node_proj(x, parts[0, :N_], parts[1, :N_],
                                nW1a[i], nW1b[i], nW2[i], nW3[i], nprm[i],
                                eW1a[i + 1], eW1b[i + 1])
            tbl = jnp.pad(tbl, ((0, N_PAD - N_), (0, 0)))
        else:
            x = _node_last(x, parts[0, :N_], parts[1, :N_],
                           nW1a[i], nW1b[i], nW2[i], nW3[i], nprm[i])
    return x


# speedup vs baseline: 4.3317x; 1.1075x over previous
import functools

import jax
import jax.numpy as jnp
from jax import lax
from jax.experimental import pallas as pl
from jax.experimental.pallas import tpu as pltpu
from jax.experimental.pallas import tpu_sc as plsc

N_ = 10000
E_ = 160000
D_ = 128
H_ = 128
HP = H_ // 2
NBLK = 9

NC = 2
NS = 16
NW = NC * NS

CH = 64
E_PAD = 163840
E_H = E_PAD // 2
PER_W = E_H // NW
NCH = PER_W // CH
CHS = 128
NCHS = PER_W // CHS
N_PAD = 10240
ROWS_PER_TILE = N_PAD // NS

_SC_MESH = plsc.VectorSubcoreMesh(
    core_axis_name="c", subcore_axis_name="s", num_cores=NC, num_subcores=NS)


def _sc_gather_body(t_h, src_h, dst_h, out_h,
                    src_v, dst_v, ba0, ba1, bb0, bb1, tsp,
                    sga0, sga1, sgb0, sgb1, swo0, swo1):
    cid = lax.axis_index("c")
    sid = lax.axis_index("s")
    wid = sid * NC + cid
    base = wid * PER_W
    srows = pl.ds(sid * ROWS_PER_TILE, ROWS_PER_TILE)
    pltpu.sync_copy(t_h.at[srows], tsp.at[srows])
    pltpu.sync_copy(src_h.at[pl.ds(base, PER_W)], src_v)
    pltpu.sync_copy(dst_h.at[pl.ds(base, PER_W)], dst_v)
    plsc.subcore_barrier()
    ba = (ba0, ba1)
    bb = (bb0, bb1)
    sga = (sga0, sga1)
    sgb = (sgb0, sgb1)
    swo = (swo0, swo1)

    def start_gather(j, b):
        pltpu.async_copy(tsp.at[src_v.at[pl.ds(j * CH, CH)]], ba[b], sga[b])
        pltpu.async_copy(tsp.at[dst_v.at[pl.ds(j * CH, CH)]], bb[b], sgb[b])

    def wait_gather(b):
        pltpu.make_async_copy(tsp.at[src_v.at[pl.ds(0, CH)]], ba[b], sga[b]).wait()
        pltpu.make_async_copy(tsp.at[dst_v.at[pl.ds(0, CH)]], bb[b], sgb[b]).wait()

    def start_write(j, b):
        pltpu.async_copy(ba[b], out_h.at[pl.ds(base + j * CH, CH)], swo[b])

    def wait_write(b):
        pltpu.make_async_copy(ba[b], out_h.at[pl.ds(base, CH)], swo[b]).wait()

    def merge(b):
        def row(r, carry):
            for k in range(4):
                lanes = pl.ds(2 * HP + 16 * k - H_ + H_, 16) if False else pl.ds(HP + 16 * k, 16)
                ba[b][r, lanes] = bb[b][r, lanes]
            return carry
        lax.fori_loop(0, CH, row, 0)

    start_gather(0, 0)

    def outer(j2, carry):
        for b in range(2):
            j = 2 * j2 + b

            @pl.when(j >= 1)
            def _():
                wait_write(1 - b)

            @pl.when(j + 1 < NCH)
            def _():
                start_gather(j + 1, 1 - b)

            wait_gather(b)
            merge(b)
            start_write(j, b)
        return carry

    lax.fori_loop(0, NCH // 2, outer, 0)
    wait_write(1)


_gather_call = pl.kernel(
    _sc_gather_body,
    out_type=jax.ShapeDtypeStruct((E_H, H_), jnp.float32),
    mesh=_SC_MESH,
    scratch_types=[
        pltpu.VMEM((PER_W,), jnp.int32),
        pltpu.VMEM((PER_W,), jnp.int32),
        pltpu.VMEM((CH, H_), jnp.float32),
        pltpu.VMEM((CH, H_), jnp.float32),
        pltpu.VMEM((CH, H_), jnp.float32),
        pltpu.VMEM((CH, H_), jnp.float32),
        pltpu.VMEM_SHARED((N_PAD, H_), jnp.float32),
    ] + [pltpu.SemaphoreType.DMA] * 6,
)


def _sc_scatter_body(ea_h, dst3_h, init_h, out_h,
                     idx_v, rb0, rb1, acc, sl0, sl1, sa0, sa1):
    cid = lax.axis_index("c")
    sid = lax.axis_index("s")
    wid = sid * NC + cid
    base = wid * PER_W
    rows = pl.ds(sid * ROWS_PER_TILE, ROWS_PER_TILE)
    pltpu.sync_copy(init_h.at[cid, rows], acc.at[rows])
    pltpu.sync_copy(dst3_h.at[wid], idx_v)
    plsc.subcore_barrier()
    rb = (rb0, rb1)
    sl = (sl0, sl1)
    sa = (sa0, sa1)

    def start_load(j, b):
        pltpu.async_copy(ea_h.at[pl.ds(base + j * CHS, CHS)], rb[b], sl[b])

    def wait_load(b):
        pltpu.make_async_copy(ea_h.at[pl.ds(base, CHS)], rb[b], sl[b]).wait()

    def start_add(j, b):
        pltpu.async_copy(rb[b], acc.at[idx_v.at[j]], sa[b], add=True)

    def wait_add(b):
        pltpu.make_async_copy(rb[b], acc.at[idx_v.at[0]], sa[b]).wait()

    start_load(0, 0)

    def outer(j2, carry):
        for b in range(2):
            j = 2 * j2 + b

            @pl.when(j >= 1)
            def _():
                wait_add(1 - b)

            @pl.when(j + 1 < NCHS)
            def _():
                start_load(j + 1, 1 - b)

            wait_load(b)
            start_add(j, b)
        return carry

    lax.fori_loop(0, NCHS // 2, outer, 0)
    wait_add(1)
    plsc.subcore_barrier()
    pltpu.sync_copy(acc.at[rows], out_h.at[cid, rows])


_scatter_call = pl.kernel(
    _sc_scatter_body,
    out_type=jax.ShapeDtypeStruct((NC, N_PAD, D_), jnp.float32),
    mesh=_SC_MESH,
    scratch_types=[
        pltpu.VMEM((NCHS, CHS), jnp.int32),
        pltpu.VMEM((CHS, D_), jnp.float32),
        pltpu.VMEM((CHS, D_), jnp.float32),
        pltpu.VMEM_SHARED((N_PAD, D_), jnp.float32),
        pltpu.SemaphoreType.DMA,
        pltpu.SemaphoreType.DMA,
        pltpu.SemaphoreType.DMA,
        pltpu.SemaphoreType.DMA,
    ],
)


TN = 1000
TE = 1024


def _pack_pairs(h):
    u = lax.bitcast_convert_type(h, jnp.uint32)
    r = (u + jnp.uint32(0x7FFF) + ((u >> 16) & jnp.uint32(1))) >> 16
    lo = r[:, :HP]
    hi = r[:, HP:]
    return lax.bitcast_convert_type(lo | (hi << 16), jnp.float32)


def _unpack_pairs(p):
    u = lax.bitcast_convert_type(p, jnp.uint32)
    rep = jnp.concatenate([u, u], axis=1)
    lane = lax.broadcasted_iota(jnp.uint32, rep.shape, 1)
    out_u = jnp.where(lane < HP, rep << 16, rep & jnp.uint32(0xFFFF0000))
    return lax.bitcast_convert_type(out_u, jnp.float32)


def _proj_body(x, w1a, w1b, out):
    xb = x[...]
    out[...] = jnp.concatenate(
        [_pack_pairs(jnp.dot(xb, w1a[...], preferred_element_type=jnp.float32)),
         _pack_pairs(jnp.dot(xb, w1b[...], preferred_element_type=jnp.float32))],
        axis=1)


@jax.jit
def _proj(x, w1a, w1b):
    return pl.pallas_call(
        _proj_body,
        grid=(N_ // TN,),
        in_specs=[
            pl.BlockSpec((TN, D_), lambda i: (i, 0)),
            pl.BlockSpec((D_, H_), lambda i: (0, 0)),
            pl.BlockSpec((D_, H_), lambda i: (0, 0)),
        ],
        out_specs=pl.BlockSpec((TN, H_), lambda i: (i, 0)),
        out_shape=jax.ShapeDtypeStruct((N_, H_), jnp.float32),
    )(x, w1a, w1b)


def _ln_2d(h, g, bt):
    m = jnp.mean(h, axis=-1, keepdims=True)
    v = jnp.mean((h - m) ** 2, axis=-1, keepdims=True)
    return (h - m) * lax.rsqrt(v + 1e-5) * g + bt


def _make_edge(base_row):
    def body(go, ea, w1c, w2, w3, prm, out):
        p = prm[...]
        b1 = p[0:1, :]
        b2 = p[1:2, :]
        b3 = p[2:3, :]
        g = p[3:4, :]
        bt = p[4:5, :]
        eab = ea[...]
        gob = go[...]
        h = (_unpack_pairs(gob[:, :HP]) + _unpack_pairs(gob[:, HP:])
             + jnp.dot(eab, w1c[...], preferred_element_type=jnp.float32) + b1)
        h = jnp.maximum(h, 0.0)
        h = jnp.maximum(jnp.dot(h, w2[...], preferred_element_type=jnp.float32) + b2, 0.0)
        h = jnp.dot(h, w3[...], preferred_element_type=jnp.float32) + b3
        ln = _ln_2d(h, g, bt)
        res = eab + ln
        if base_row + E_H > E_:
            row = (base_row + TE * pl.program_id(0)
                   + lax.broadcasted_iota(jnp.int32, (TE, 1), 0))
            res = jnp.where(row < E_, res, 0.0)
        out[...] = res

    @jax.jit
    def call(go, ea, w1c, w2, w3, prm):
        return pl.pallas_call(
            body,
            grid=(E_H // TE,),
            in_specs=[
                pl.BlockSpec((TE, H_), lambda i: (i, 0)),
                pl.BlockSpec((TE, D_), lambda i: (i, 0)),
                pl.BlockSpec((D_, H_), lambda i: (0, 0)),
                pl.BlockSpec((H_, H_), lambda i: (0, 0)),
                pl.BlockSpec((H_, D_), lambda i: (0, 0)),
                pl.BlockSpec((8, 128), lambda i: (0, 0)),
            ],
            out_specs=pl.BlockSpec((TE, D_), lambda i: (i, 0)),
            out_shape=jax.ShapeDtypeStruct((E_H, D_), jnp.float32),
        )(go, ea, w1c, w2, w3, prm)
    return call


_edge_h = (_make_edge(0), _make_edge(E_H))


def _node_body(x, a0, a1, w1a, w1b, w2, w3, prm, ew1a, ew1b, out, outt):
    p = prm[...]
    b1 = p[0:1, :]
    b2 = p[1:2, :]
    b3 = p[2:3, :]
    g = p[3:4, :]
    bt = p[4:5, :]
    xb = x[...]
    agg = a0[...] + a1[...]
    h = (jnp.dot(xb, w1a[...], preferred_element_type=jnp.float32)
         + jnp.dot(agg, w1b[...], preferred_element_type=jnp.float32) + b1)
    h = jnp.maximum(h, 0.0)
    h = jnp.maximum(jnp.dot(h, w2[...], preferred_element_type=jnp.float32) + b2, 0.0)
    h = jnp.dot(h, w3[...], preferred_element_type=jnp.float32) + b3
    xn = xb + _ln_2d(h, g, bt)
    out[...] = xn
    outt[...] = jnp.concatenate(
        [_pack_pairs(jnp.dot(xn, ew1a[...], preferred_element_type=jnp.float32)),
         _pack_pairs(jnp.dot(xn, ew1b[...], preferred_element_type=jnp.float32))],
        axis=1)


@jax.jit
def _node_proj(x, a0, a1, w1a, w1b, w2, w3, prm, ew1a, ew1b):
    return pl.pallas_call(
        _node_body,
        grid=(N_ // TN,),
        in_specs=[
            pl.BlockSpec((TN, D_), lambda i: (i, 0)),
            pl.BlockSpec((TN, D_), lambda i: (i, 0)),
            pl.BlockSpec((TN, D_), lambda i: (i, 0)),
            pl.BlockSpec((D_, H_), lambda i: (0, 0)),
            pl.BlockSpec((D_, H_), lambda i: (0, 0)),
            pl.BlockSpec((H_, H_), lambda i: (0, 0)),
            pl.BlockSpec((H_, D_), lambda i: (0, 0)),
            pl.BlockSpec((8, 128), lambda i: (0, 0)),
            pl.BlockSpec((D_, H_), lambda i: (0, 0)),
            pl.BlockSpec((D_, H_), lambda i: (0, 0)),
        ],
        out_specs=[pl.BlockSpec((TN, D_), lambda i: (i, 0)),
                   pl.BlockSpec((TN, H_), lambda i: (i, 0))],
        out_shape=[jax.ShapeDtypeStruct((N_, D_), jnp.float32),
                   jax.ShapeDtypeStruct((N_, H_), jnp.float32)],
    )(x, a0, a1, w1a, w1b, w2, w3, prm, ew1a, ew1b)


def _node_last_body(x, a0, a1, w1a, w1b, w2, w3, prm, out):
    p = prm[...]
    b1 = p[0:1, :]
    b2 = p[1:2, :]
    b3 = p[2:3, :]
    g = p[3:4, :]
    bt = p[4:5, :]
    xb = x[...]
    agg = a0[...] + a1[...]
    h = (jnp.dot(xb, w1a[...], preferred_element_type=jnp.float32)
         + jnp.dot(agg, w1b[...], preferred_element_type=jnp.float32) + b1)
    h = jnp.maximum(h, 0.0)
    h = jnp.maximum(jnp.dot(h, w2[...], preferred_element_type=jnp.float32) + b2, 0.0)
    h = jnp.dot(h, w3[...], preferred_element_type=jnp.float32) + b3
    out[...] = xb + _ln_2d(h, g, bt)


@jax.jit
def _node_last(x, a0, a1, w1a, w1b, w2, w3, prm):
    return pl.pallas_call(
        _node_last_body,
        grid=(N_ // TN,),
        in_specs=[
            pl.BlockSpec((TN, D_), lambda i: (i, 0)),
            pl.BlockSpec((TN, D_), lambda i: (i, 0)),
            pl.BlockSpec((TN, D_), lambda i: (i, 0)),
            pl.BlockSpec((D_, H_), lambda i: (0, 0)),
            pl.BlockSpec((D_, H_), lambda i: (0, 0)),
            pl.BlockSpec((H_, H_), lambda i: (0, 0)),
            pl.BlockSpec((H_, D_), lambda i: (0, 0)),
            pl.BlockSpec((8, 128), lambda i: (0, 0)),
        ],
        out_specs=pl.BlockSpec((TN, D_), lambda i: (i, 0)),
        out_shape=jax.ShapeDtypeStruct((N_, D_), jnp.float32),
    )(x, a0, a1, w1a, w1b, w2, w3, prm)


def _pack_params(b1, b2, b3, g, bt):
    p = jnp.stack([b1, b2, b3, g, bt], axis=1)
    return jnp.pad(p, ((0, 0), (0, 3), (0, 0)))


def kernel(x, edge_index, edge_attr, eW1, eb1, eW2, eb2, eW3, eb3, eg, ebt,
           nW1, nb1, nW2, nb2, nW3, nb3, ng, nbt):
    pad = E_PAD - E_
    src_p = jnp.pad(edge_index[0], (0, pad))
    dst_p = jnp.pad(edge_index[1], (0, pad))
    ea_p = jnp.pad(edge_attr, ((0, pad), (0, 0)))
    src_h = (src_p[:E_H], src_p[E_H:])
    dst_h = (dst_p[:E_H], dst_p[E_H:])
    dst3_h = (dst_p[:E_H].reshape(NW, NCHS, CHS), dst_p[E_H:].reshape(NW, NCHS, CHS))
    ea_h = [ea_p[:E_H], ea_p[E_H:]]
    zeros3 = jnp.zeros((NC, N_PAD, D_), jnp.float32)

    eW1a = eW1[:, :D_, :]
    eW1b = eW1[:, D_:2 * D_, :]
    eW1c = eW1[:, 2 * D_:, :]
    nW1a = nW1[:, :D_, :]
    nW1b = nW1[:, D_:, :]
    eprm = _pack_params(eb1, eb2, eb3, eg, ebt)
    nprm = _pack_params(nb1, nb2, nb3, ng, nbt)

    tbl = jnp.pad(_proj(x, eW1a[0], eW1b[0]), ((0, N_PAD - N_), (0, 0)))
    for i in range(NBLK):
        go0 = _gather_call(tbl, src_h[0], dst_h[0])
        go1 = _gather_call(tbl, src_h[1], dst_h[1])
        ea_h[0] = _edge_h[0](go0, ea_h[0], eW1c[i], eW2[i], eW3[i], eprm[i])
        parts0 = _scatter_call(ea_h[0], dst3_h[0], zeros3)
        ea_h[1] = _edge_h[1](go1, ea_h[1], eW1c[i], eW2[i], eW3[i], eprm[i])
        parts = _scatter_call(ea_h[1], dst3_h[1], parts0)
        if i + 1 < NBLK:
            x, tbl = _node_proj(x, parts[0, :N_], parts[1, :N_],
                                nW1a[i], nW1b[i], nW2[i], nW3[i], nprm[i],
                                eW1a[i + 1], eW1b[i + 1])
            tbl = jnp.pad(tbl, ((0, N_PAD - N_), (0, 0)))
        else:
            x = _node_last(x, parts[0, :N_], parts[1, :N_],
                           nW1a[i], nW1b[i], nW2[i], nW3[i], nprm[i])
    return x


# speedup vs baseline: 4.3318x; 1.0000x over previous
import jax
import jax.numpy as jnp
from jax import lax
from jax.experimental import pallas as pl
from jax.experimental.pallas import tpu as pltpu
from jax.experimental.pallas import tpu_sc as plsc

N_ = 10000
E_ = 160000
D_ = 128
H_ = 128
HP = H_ // 2
NBLK = 9

NC = 2
NS = 16
NW = NC * NS

CH = 64
E_PAD = 163840
E_H = E_PAD // 2
PER_W = E_H // NW
NCH = PER_W // CH
CHS = 128
NCHS = PER_W // CHS
N_PAD = 10240
ROWS_PER_TILE = N_PAD // NS

_SC_MESH = plsc.VectorSubcoreMesh(
    core_axis_name="c", subcore_axis_name="s", num_cores=NC, num_subcores=NS)


def _sc_gather_body(t_h, src_h, dst_h, out_h,
                    src_v, dst_v, ba0, ba1, bb0, bb1, tsp,
                    sga0, sga1, sgb0, sgb1, swo0, swo1):
    cid = lax.axis_index("c")
    sid = lax.axis_index("s")
    wid = sid * NC + cid
    base = wid * PER_W
    srows = pl.ds(sid * ROWS_PER_TILE, ROWS_PER_TILE)
    pltpu.sync_copy(t_h.at[srows], tsp.at[srows])
    pltpu.sync_copy(src_h.at[pl.ds(base, PER_W)], src_v)
    pltpu.sync_copy(dst_h.at[pl.ds(base, PER_W)], dst_v)
    plsc.subcore_barrier()
    ba = (ba0, ba1)
    bb = (bb0, bb1)
    sga = (sga0, sga1)
    sgb = (sgb0, sgb1)
    swo = (swo0, swo1)

    def start_gather(j, b):
        pltpu.async_copy(tsp.at[src_v.at[pl.ds(j * CH, CH)]], ba[b], sga[b])
        pltpu.async_copy(tsp.at[dst_v.at[pl.ds(j * CH, CH)]], bb[b], sgb[b])

    def wait_gather(b):
        pltpu.make_async_copy(tsp.at[src_v.at[pl.ds(0, CH)]], ba[b], sga[b]).wait()
        pltpu.make_async_copy(tsp.at[dst_v.at[pl.ds(0, CH)]], bb[b], sgb[b]).wait()

    def start_write(j, b):
        pltpu.async_copy(ba[b], out_h.at[pl.ds(base + j * CH, CH)], swo[b])

    def wait_write(b):
        pltpu.make_async_copy(ba[b], out_h.at[pl.ds(base, CH)], swo[b]).wait()

    def merge(b):
        def row(r, carry):
            for k in range(4):
                lanes = pl.ds(HP + 16 * k, 16)
                ba[b][r, lanes] = bb[b][r, lanes]
            return carry
        lax.fori_loop(0, CH, row, 0)

    start_gather(0, 0)

    def outer(j2, carry):
        for b in range(2):
            j = 2 * j2 + b

            @pl.when(j >= 1)
            def _():
                wait_write(1 - b)

            @pl.when(j + 1 < NCH)
            def _():
                start_gather(j + 1, 1 - b)

            wait_gather(b)
            merge(b)
            start_write(j, b)
        return carry

    lax.fori_loop(0, NCH // 2, outer, 0)
    wait_write(1)


_gather_call = pl.kernel(
    _sc_gather_body,
    out_type=jax.ShapeDtypeStruct((E_H, H_), jnp.float32),
    mesh=_SC_MESH,
    scratch_types=[
        pltpu.VMEM((PER_W,), jnp.int32),
        pltpu.VMEM((PER_W,), jnp.int32),
        pltpu.VMEM((CH, H_), jnp.float32),
        pltpu.VMEM((CH, H_), jnp.float32),
        pltpu.VMEM((CH, H_), jnp.float32),
        pltpu.VMEM((CH, H_), jnp.float32),
        pltpu.VMEM_SHARED((N_PAD, H_), jnp.float32),
    ] + [pltpu.SemaphoreType.DMA] * 6,
)


def _sc_scatter_body(ea_h, dst3_h, init_h, out_h,
                     idx_v, rb0, rb1, acc, sl0, sl1, sa0, sa1):
    cid = lax.axis_index("c")
    sid = lax.axis_index("s")
    wid = sid * NC + cid
    base = wid * PER_W
    rows = pl.ds(sid * ROWS_PER_TILE, ROWS_PER_TILE)
    pltpu.sync_copy(init_h.at[cid, rows], acc.at[rows])
    pltpu.sync_copy(dst3_h.at[wid], idx_v)
    plsc.subcore_barrier()
    rb = (rb0, rb1)
    sl = (sl0, sl1)
    sa = (sa0, sa1)

    def start_load(j, b):
        pltpu.async_copy(ea_h.at[pl.ds(base + j * CHS, CHS)], rb[b], sl[b])

    def wait_load(b):
        pltpu.make_async_copy(ea_h.at[pl.ds(base, CHS)], rb[b], sl[b]).wait()

    def start_add(j, b):
        pltpu.async_copy(rb[b], acc.at[idx_v.at[j]], sa[b], add=True)

    def wait_add(b):
        pltpu.make_async_copy(rb[b], acc.at[idx_v.at[0]], sa[b]).wait()

    start_load(0, 0)

    def outer(j2, carry):
        for b in range(2):
            j = 2 * j2 + b

            @pl.when(j >= 1)
            def _():
                wait_add(1 - b)

            @pl.when(j + 1 < NCHS)
            def _():
                start_load(j + 1, 1 - b)

            wait_load(b)
            start_add(j, b)
        return carry

    lax.fori_loop(0, NCHS // 2, outer, 0)
    wait_add(1)
    plsc.subcore_barrier()
    pltpu.sync_copy(acc.at[rows], out_h.at[cid, rows])


_scatter_call = pl.kernel(
    _sc_scatter_body,
    out_type=jax.ShapeDtypeStruct((NC, N_PAD, D_), jnp.float32),
    mesh=_SC_MESH,
    scratch_types=[
        pltpu.VMEM((NCHS, CHS), jnp.int32),
        pltpu.VMEM((CHS, D_), jnp.float32),
        pltpu.VMEM((CHS, D_), jnp.float32),
        pltpu.VMEM_SHARED((N_PAD, D_), jnp.float32),
        pltpu.SemaphoreType.DMA,
        pltpu.SemaphoreType.DMA,
        pltpu.SemaphoreType.DMA,
        pltpu.SemaphoreType.DMA,
    ],
)


TN = 1000
TE = 1024


def _pack_pairs(h):
    u = lax.bitcast_convert_type(h, jnp.uint32)
    r = (u + jnp.uint32(0x7FFF) + ((u >> 16) & jnp.uint32(1))) >> 16
    lo = r[:, :HP]
    hi = r[:, HP:]
    return lax.bitcast_convert_type(lo | (hi << 16), jnp.float32)


def _unpack_pairs(p):
    u = lax.bitcast_convert_type(p, jnp.uint32)
    rep = jnp.concatenate([u, u], axis=1)
    lane = lax.broadcasted_iota(jnp.uint32, rep.shape, 1)
    out_u = jnp.where(lane < HP, rep << 16, rep & jnp.uint32(0xFFFF0000))
    return lax.bitcast_convert_type(out_u, jnp.float32)


def _proj_body(x, w1a, w1b, out):
    xb = x[...]
    out[...] = jnp.concatenate(
        [_pack_pairs(jnp.dot(xb, w1a[...], preferred_element_type=jnp.float32)),
         _pack_pairs(jnp.dot(xb, w1b[...], preferred_element_type=jnp.float32))],
        axis=1)


@jax.jit
def _proj(x, w1a, w1b):
    return pl.pallas_call(
        _proj_body,
        grid=(N_ // TN,),
        in_specs=[
            pl.BlockSpec((TN, D_), lambda i: (i, 0)),
            pl.BlockSpec((D_, H_), lambda i: (0, 0)),
            pl.BlockSpec((D_, H_), lambda i: (0, 0)),
        ],
        out_specs=pl.BlockSpec((TN, H_), lambda i: (i, 0)),
        out_shape=jax.ShapeDtypeStruct((N_, H_), jnp.float32),
    )(x, w1a, w1b)


def _ln_2d(h, g, bt):
    m = jnp.mean(h, axis=-1, keepdims=True)
    v = jnp.mean((h - m) ** 2, axis=-1, keepdims=True)
    return (h - m) * lax.rsqrt(v + 1e-5) * g + bt


def _make_edge(base_row):
    def body(go, ea, w1c, w2, w3, prm, out):
        p = prm[...]
        b1 = p[0:1, :]
        b2 = p[1:2, :]
        b3 = p[2:3, :]
        g = p[3:4, :]
        bt = p[4:5, :]
        eab = ea[...]
        gob = go[...]
        h = (_unpack_pairs(gob[:, :HP]) + _unpack_pairs(gob[:, HP:])
             + jnp.dot(eab, w1c[...], preferred_element_type=jnp.float32) + b1)
        h = jnp.maximum(h, 0.0)
        h = jnp.maximum(jnp.dot(h, w2[...], preferred_element_type=jnp.float32) + b2, 0.0)
        h = jnp.dot(h, w3[...], preferred_element_type=jnp.float32) + b3
        ln = _ln_2d(h, g, bt)
        res = eab + ln
        if base_row + E_H > E_:
            row = (base_row + TE * pl.program_id(0)
                   + lax.broadcasted_iota(jnp.int32, (TE, 1), 0))
            res = jnp.where(row < E_, res, 0.0)
        out[...] = res

    @jax.jit
    def call(go, ea, w1c, w2, w3, prm):
        return pl.pallas_call(
            body,
            grid=(E_H // TE,),
            in_specs=[
                pl.BlockSpec((TE, H_), lambda i: (i, 0)),
                pl.BlockSpec((TE, D_), lambda i: (i, 0)),
                pl.BlockSpec((D_, H_), lambda i: (0, 0)),
                pl.BlockSpec((H_, H_), lambda i: (0, 0)),
                pl.BlockSpec((H_, D_), lambda i: (0, 0)),
                pl.BlockSpec((8, 128), lambda i: (0, 0)),
            ],
            out_specs=pl.BlockSpec((TE, D_), lambda i: (i, 0)),
            out_shape=jax.ShapeDtypeStruct((E_H, D_), jnp.float32),
        )(go, ea, w1c, w2, w3, prm)
    return call


_edge_h = (_make_edge(0), _make_edge(E_H))


def _node_body(x, a0, a1, w1a, w1b, w2, w3, prm, ew1a, ew1b, out, outt):
    p = prm[...]
    b1 = p[0:1, :]
    b2 = p[1:2, :]
    b3 = p[2:3, :]
    g = p[3:4, :]
    bt = p[4:5, :]
    xb = x[...]
    agg = a0[...] + a1[...]
    h = (jnp.dot(xb, w1a[...], preferred_element_type=jnp.float32)
         + jnp.dot(agg, w1b[...], preferred_element_type=jnp.float32) + b1)
    h = jnp.maximum(h, 0.0)
    h = jnp.maximum(jnp.dot(h, w2[...], preferred_element_type=jnp.float32) + b2, 0.0)
    h = jnp.dot(h, w3[...], preferred_element_type=jnp.float32) + b3
    xn = xb + _ln_2d(h, g, bt)
    out[...] = xn
    outt[...] = jnp.concatenate(
        [_pack_pairs(jnp.dot(xn, ew1a[...], preferred_element_type=jnp.float32)),
         _pack_pairs(jnp.dot(xn, ew1b[...], preferred_element_type=jnp.float32))],
        axis=1)


@jax.jit
def _node_proj(x, a0, a1, w1a, w1b, w2, w3, prm, ew1a, ew1b):
    return pl.pallas_call(
        _node_body,
        grid=(N_ // TN,),
        in_specs=[
            pl.BlockSpec((TN, D_), lambda i: (i, 0)),
            pl.BlockSpec((TN, D_), lambda i: (i, 0)),
            pl.BlockSpec((TN, D_), lambda i: (i, 0)),
            pl.BlockSpec((D_, H_), lambda i: (0, 0)),
            pl.BlockSpec((D_, H_), lambda i: (0, 0)),
            pl.BlockSpec((H_, H_), lambda i: (0, 0)),
            pl.BlockSpec((H_, D_), lambda i: (0, 0)),
            pl.BlockSpec((8, 128), lambda i: (0, 0)),
            pl.BlockSpec((D_, H_), lambda i: (0, 0)),
            pl.BlockSpec((D_, H_), lambda i: (0, 0)),
        ],
        out_specs=[pl.BlockSpec((TN, D_), lambda i: (i, 0)),
                   pl.BlockSpec((TN, H_), lambda i: (i, 0))],
        out_shape=[jax.ShapeDtypeStruct((N_, D_), jnp.float32),
                   jax.ShapeDtypeStruct((N_, H_), jnp.float32)],
    )(x, a0, a1, w1a, w1b, w2, w3, prm, ew1a, ew1b)


def _node_last_body(x, a0, a1, w1a, w1b, w2, w3, prm, out):
    p = prm[...]
    b1 = p[0:1, :]
    b2 = p[1:2, :]
    b3 = p[2:3, :]
    g = p[3:4, :]
    bt = p[4:5, :]
    xb = x[...]
    agg = a0[...] + a1[...]
    h = (jnp.dot(xb, w1a[...], preferred_element_type=jnp.float32)
         + jnp.dot(agg, w1b[...], preferred_element_type=jnp.float32) + b1)
    h = jnp.maximum(h, 0.0)
    h = jnp.maximum(jnp.dot(h, w2[...], preferred_element_type=jnp.float32) + b2, 0.0)
    h = jnp.dot(h, w3[...], preferred_element_type=jnp.float32) + b3
    out[...] = xb + _ln_2d(h, g, bt)


@jax.jit
def _node_last(x, a0, a1, w1a, w1b, w2, w3, prm):
    return pl.pallas_call(
        _node_last_body,
        grid=(N_ // TN,),
        in_specs=[
            pl.BlockSpec((TN, D_), lambda i: (i, 0)),
            pl.BlockSpec((TN, D_), lambda i: (i, 0)),
            pl.BlockSpec((TN, D_), lambda i: (i, 0)),
            pl.BlockSpec((D_, H_), lambda i: (0, 0)),
            pl.BlockSpec((D_, H_), lambda i: (0, 0)),
            pl.BlockSpec((H_, H_), lambda i: (0, 0)),
            pl.BlockSpec((H_, D_), lambda i: (0, 0)),
            pl.BlockSpec((8, 128), lambda i: (0, 0)),
        ],
        out_specs=pl.BlockSpec((TN, D_), lambda i: (i, 0)),
        out_shape=jax.ShapeDtypeStruct((N_, D_), jnp.float32),
    )(x, a0, a1, w1a, w1b, w2, w3, prm)


def _pack_params(b1, b2, b3, g, bt):
    p = jnp.stack([b1, b2, b3, g, bt], axis=1)
    return jnp.pad(p, ((0, 0), (0, 3), (0, 0)))


def kernel(x, edge_index, edge_attr, eW1, eb1, eW2, eb2, eW3, eb3, eg, ebt,
           nW1, nb1, nW2, nb2, nW3, nb3, ng, nbt):
    pad = E_PAD - E_
    src_p = jnp.pad(edge_index[0], (0, pad))
    dst_p = jnp.pad(edge_index[1], (0, pad))
    ea_p = jnp.pad(edge_attr, ((0, pad), (0, 0)))
    src_h = (src_p[:E_H], src_p[E_H:])
    dst_h = (dst_p[:E_H], dst_p[E_H:])
    dst3_h = (dst_p[:E_H].reshape(NW, NCHS, CHS), dst_p[E_H:].reshape(NW, NCHS, CHS))
    ea_h = [ea_p[:E_H], ea_p[E_H:]]
    zeros3 = jnp.zeros((NC, N_PAD, D_), jnp.float32)

    eW1a = eW1[:, :D_, :]
    eW1b = eW1[:, D_:2 * D_, :]
    eW1c = eW1[:, 2 * D_:, :]
    nW1a = nW1[:, :D_, :]
    nW1b = nW1[:, D_:, :]
    eprm = _pack_params(eb1, eb2, eb3, eg, ebt)
    nprm = _pack_params(nb1, nb2, nb3, ng, nbt)

    tbl = jnp.pad(_proj(x, eW1a[0], eW1b[0]), ((0, N_PAD - N_), (0, 0)))
    for i in range(NBLK):
        go0 = _gather_call(tbl, src_h[0], dst_h[0])
        go1 = _gather_call(tbl, src_h[1], dst_h[1])
        ea_h[0] = _edge_h[0](go0, ea_h[0], eW1c[i], eW2[i], eW3[i], eprm[i])
        parts0 = _scatter_call(ea_h[0], dst3_h[0], zeros3)
        ea_h[1] = _edge_h[1](go1, ea_h[1], eW1c[i], eW2[i], eW3[i], eprm[i])
        parts = _scatter_call(ea_h[1], dst3_h[1], parts0)
        if i + 1 < NBLK:
            x, tbl = _node_proj(x, parts[0, :N_], parts[1, :N_],
                                nW1a[i], nW1b[i], nW2[i], nW3[i], nprm[i],
                                eW1a[i + 1], eW1b[i + 1])
            tbl = jnp.pad(tbl, ((0, N_PAD - N_), (0, 0)))
        else:
            x = _node_last(x, parts[0, :N_], parts[1, :N_],
                           nW1a[i], nW1b[i], nW2[i], nW3[i], nprm[i])
    return x
